# baseline (speedup 1.0000x reference)
import functools

import jax
import jax.numpy as jnp
from jax import lax
from jax.experimental import pallas as pl
from jax.experimental.pallas import tpu as pltpu
from jax.experimental.pallas import tpu_sc as plsc

N_G = 10000
E_G = 160000
E_LG = 480000
N_LG = 160000
HID = 128
L = 7
NUM_GRAPHS = 100

NW = 32
D_CHUNK = 2048
N_CHUNK = 80
ACC_R = D_CHUNK + 128
NGP = 10240

_mesh = plsc.VectorSubcoreMesh(core_axis_name="c", subcore_axis_name="s")


def _sc_gather(table, idx):
    B = idx.shape[0]
    per_w = B // NW
    assert per_w * NW == B and per_w % 8 == 0
    nblk = -(-per_w // 128)
    assert nblk >= 8
    Bp = B + nblk * 128 - per_w
    idx_p = jnp.zeros((Bp,), jnp.int32).at[:B].set(idx)
    NBUF = 6
    ngrp = nblk // NBUF

    @functools.partial(
        pl.kernel, mesh=_mesh,
        out_type=jax.ShapeDtypeStruct((Bp, HID), jnp.float32),
        scratch_types=[
            pltpu.VMEM((nblk * 128,), jnp.int32),
        ] + [pltpu.VMEM((128, HID), jnp.float32) for _ in range(NBUF)]
          + [pltpu.SemaphoreType.DMA for _ in range(2 * NBUF)],
    )
    def k(tab, ix, out, ixall, *bufs_sems):
        rows = bufs_sems[:NBUF]
        gsem = bufs_sems[NBUF:2 * NBUF]
        osem = bufs_sems[2 * NBUF:]
        wid = lax.axis_index("s") * 2 + lax.axis_index("c")
        base = wid * per_w
        pltpu.sync_copy(ix.at[pl.ds(base, nblk * 128)], ixall)

        def g_start(i, b):
            pltpu.async_copy(tab.at[ixall.at[pl.ds(i * 128, 128)]],
                             rows[b], gsem[b])

        def w_start(i, b):
            pltpu.async_copy(rows[b], out.at[pl.ds(base + i * 128, 128)],
                             osem[b])

        g_wait = [
            pltpu.make_async_copy(tab.at[ixall.at[pl.ds(0, 128)]],
                                  rows[b], gsem[b])
            for b in range(NBUF)
        ]
        o_wait = [
            pltpu.make_async_copy(rows[b], out.at[pl.ds(0, 128)], osem[b])
            for b in range(NBUF)
        ]

        for b in range(NBUF - 2):
            g_start(b, b)

        def step(j, b, bprev, traced):
            g_wait[b].wait()
            w_start(j, b)
            k2 = j + NBUF - 2
            if traced:
                @pl.when(k2 < nblk)
                def _():
                    @pl.when(j >= 2)
                    def _():
                        o_wait[bprev].wait()

                    g_start(k2, bprev)
            else:
                if k2 < nblk:
                    if j >= 2:
                        o_wait[bprev].wait()
                    g_start(k2, bprev)

        def grp(j, carry):
            for b in range(NBUF):
                i = j * NBUF + b
                step(i, b, (b - 2) % NBUF, True)
            return carry

        lax.fori_loop(0, ngrp, grp, 0)
        for i in range(ngrp * NBUF, nblk):
            step(i, i % NBUF, (i - 2) % NBUF, False)
        for i in range(nblk - NBUF, nblk):
            o_wait[i % NBUF].wait()

    return k(table, idx_p)


def _sc_scatter_softmax(ema, dst_s, ofs_pad):
    ROWS_T = ACC_R // 16
    FIN_T = D_CHUNK // 16

    @functools.partial(
        pl.kernel, mesh=_mesh,
        out_type=jax.ShapeDtypeStruct((N_LG, HID), jnp.float32),
        scratch_types=[
            pltpu.VMEM((128,), jnp.int32),
            pltpu.VMEM((128,), jnp.int32),
            pltpu.VMEM((128,), jnp.int32),
            pltpu.VMEM((128,), jnp.int32),
            pltpu.VMEM((256, HID), jnp.float32),
            pltpu.VMEM((256, HID), jnp.float32),
            pltpu.VMEM((64, HID), jnp.float32),
            pltpu.VMEM((96,), jnp.int32),
            pltpu.VMEM_SHARED((ACC_R, HID), jnp.float32),
            pltpu.VMEM_SHARED((ACC_R, HID), jnp.float32),
            pltpu.SemaphoreType.DMA,
            pltpu.SemaphoreType.DMA,
        ],
    )
    def k(ema_h, dst_h, ofs_h, out_h,
          draw0, draw1, dloc0, dloc1, pay0, pay1, zb, ofsv,
          acc_s, acc_a, sem0, sem1):
        cid = lax.axis_index("c")
        sid = lax.axis_index("s")
        draw = (draw0, draw1)
        dloc = (dloc0, dloc1)
        pay = (pay0, pay1)
        psem = (sem0, sem1)
        pltpu.sync_copy(ofs_h, ofsv)

        zero16 = jnp.zeros((16,), jnp.float32)

        def zrow(r, carry):
            for k2 in range(8):
                zb[r, pl.ds(k2 * 16, 16)] = zero16
            return carry

        lax.fori_loop(0, 64, zrow, 0)

        pwait = [
            pltpu.make_async_copy(ema_h.at[pl.ds(0, 256)], pay[b], psem[b])
            for b in range(2)
        ]

        def chunk_body(c, carry):
            z0 = sid * ROWS_T
            for acc in (acc_s, acc_a):
                pltpu.sync_copy(zb, acc.at[pl.ds(z0, 64)])
                pltpu.sync_copy(zb, acc.at[pl.ds(z0 + 64, 64)])
                pltpu.sync_copy(zb.at[pl.ds(0, ROWS_T - 128)],
                                acc.at[pl.ds(z0 + 128, ROWS_T - 128)])
            plsc.subcore_barrier()

            ov = ofsv[pl.ds(c, 16)]
            start = ov[0]
            end = ov[1]
            start_al = (start // 128) * 128
            nblk = (end - start_al + 127) // 128
            nmine = jnp.maximum((nblk - sid + 15) // 16, 0)
            cbase = c * D_CHUNK

            def pay_start(i, b):
                eofs = start_al + (sid + i * 16) * 128
                pltpu.async_copy(ema_h.at[pl.ds(2 * eofs, 256)],
                                 pay[b], psem[b])

            for b in range(2):
                @pl.when(b < nmine)
                def _():
                    pay_start(b, b)

            def blk_body(j, carry2):
                for b in range(2):
                    i = j * 2 + b

                    @pl.when(i < nmine)
                    def _():
                        eofs = start_al + (sid + i * 16) * 128
                        pltpu.sync_copy(dst_h.at[pl.ds(eofs, 128)], draw[b])
                        for k2 in range(8):
                            v = draw[b][pl.ds(k2 * 16, 16)]
                            lv = v - cbase
                            ok = (lv >= 0) & (lv < D_CHUNK)
                            dloc[b][pl.ds(k2 * 16, 16)] = jnp.where(
                                ok, lv, D_CHUNK)
                        pwait[b].wait()
                        pltpu.sync_copy(pay[b].at[pl.ds(0, 128)],
                                        acc_s.at[dloc[b]], add=True)
                        pltpu.sync_copy(pay[b].at[pl.ds(128, 128)],
                                        acc_a.at[dloc[b]], add=True)

                        @pl.when(i + 2 < nmine)
                        def _():
                            pay_start(i + 2, b)
                return carry2

            lax.fori_loop(0, (nmine + 1) // 2, blk_body, 0)
            plsc.subcore_barrier()

            for half in range(1):
                r0 = sid * FIN_T + half * 128

                @pl.when(cbase + r0 < N_LG)
                def _():
                    pltpu.sync_copy(acc_s.at[pl.ds(r0, 128)],
                                    pay0.at[pl.ds(0, 128)])
                    pltpu.sync_copy(acc_a.at[pl.ds(r0, 128)],
                                    pay0.at[pl.ds(128, 128)])

                    def fin(r, carry3):
                        for k2 in range(8):
                            s_ = pay0[r, pl.ds(k2 * 16, 16)]
                            a = pay0[128 + r, pl.ds(k2 * 16, 16)]
                            pay0[r, pl.ds(k2 * 16, 16)] = a / (s_ + 1e-16)
                        return carry3

                    lax.fori_loop(0, 128, fin, 0)
                    pltpu.sync_copy(pay0.at[pl.ds(0, 128)],
                                    out_h.at[pl.ds(cbase + r0, 128)])
            plsc.subcore_barrier()
            return carry

        lax.fori_loop(cid * (N_CHUNK // 2), (cid + 1) * (N_CHUNK // 2),
                      chunk_body, 0)

    return k(ema, dst_s, ofs_pad)


def _sc_scatter_final(h, dst_g):
    per_w = E_G // NW
    nfull = per_w // 128
    tail = per_w - nfull * 128
    ROWS_T = NGP // 16

    @functools.partial(
        pl.kernel, mesh=_mesh,
        out_type=jax.ShapeDtypeStruct((2, NGP, HID), jnp.float32),
        scratch_types=[
            pltpu.VMEM((128,), jnp.int32),
            pltpu.VMEM((128, HID), jnp.float32),
            pltpu.VMEM((128, HID), jnp.float32),
            pltpu.VMEM_SHARED((NGP, HID), jnp.float32),
            pltpu.SemaphoreType.DMA,
        ],
    )
    def k(h_h, dg_h, out_h, draw, pay, zb, acc, sem):
        cid = lax.axis_index("c")
        sid = lax.axis_index("s")
        wid = sid * 2 + cid
        base = wid * per_w

        zero16 = jnp.zeros((16,), jnp.float32)
        trash16 = jnp.full((16,), NGP - 1, jnp.int32)

        def zrow(r, carry):
            for k2 in range(8):
                zb[r, pl.ds(k2 * 16, 16)] = zero16
            return carry

        lax.fori_loop(0, 128, zrow, 0)

        z0 = sid * ROWS_T
        for q in range(5):
            pltpu.sync_copy(zb.at[pl.ds(0, 128)],
                            acc.at[pl.ds(z0 + q * 128, 128)])
        plsc.subcore_barrier()

        def body(i, carry):
            off = base + i * 128
            pltpu.sync_copy(dg_h.at[pl.ds(off, 128)], draw)
            pltpu.sync_copy(h_h.at[pl.ds(off, 128)], pay)
            pltpu.sync_copy(pay, acc.at[draw], add=True)
            return carry

        lax.fori_loop(0, nfull, body, 0)

        for k2 in range(8):
            draw[pl.ds(k2 * 16, 16)] = trash16

        def zpay(r, carry):
            for k2 in range(8):
                pay[r, pl.ds(k2 * 16, 16)] = zero16
            return carry

        lax.fori_loop(0, 128, zpay, 0)
        toff = base + nfull * 128
        pltpu.sync_copy(dg_h.at[pl.ds(toff, tail)], draw.at[pl.ds(0, tail)])
        pltpu.sync_copy(h_h.at[pl.ds(toff, tail)], pay.at[pl.ds(0, tail)])
        pltpu.sync_copy(pay, acc.at[draw], add=True)
        plsc.subcore_barrier()

        for q in range(5):
            pltpu.sync_copy(acc.at[pl.ds(z0 + q * 128, 128)],
                            out_h.at[cid, pl.ds(z0 + q * 128, 128)])

    return k(h, dst_g)


def _tc_enc(x, W, b):
    def body(x_ref, w_ref, b_ref, o_ref):
        o_ref[...] = jnp.dot(x_ref[...], w_ref[...],
                             preferred_element_type=jnp.float32) + b_ref[...]

    return pl.pallas_call(
        body,
        grid=(10,),
        in_specs=[
            pl.BlockSpec((1000, HID), lambda i: (i, 0)),
            pl.BlockSpec((HID, HID), lambda i: (0, 0)),
            pl.BlockSpec((1, HID), lambda i: (0, 0)),
        ],
        out_specs=pl.BlockSpec((1000, HID), lambda i: (i, 0)),
        out_shape=jax.ShapeDtypeStruct((N_G, HID), jnp.float32),
    )(x, W, b.reshape(1, HID))


def _tc_msg(hsd, ea, xl, edb, W1, W2, W3, W4, bm, Wnb, bnb):
    BLK = 2000
    OFF = E_G // BLK

    def body(hs_r, hd_r, ea_r, xl_r, edb_r, w1, w2, w3, w4, bm_r, wnb, bnb_r,
             h_ref, nb_ref):
        acc = jnp.dot(hs_r[...], w1[...], preferred_element_type=jnp.float32)
        acc += jnp.dot(hd_r[...], w2[...], preferred_element_type=jnp.float32)
        acc += jnp.dot(ea_r[...], w3[...], preferred_element_type=jnp.float32)
        acc += jnp.dot(xl_r[...], w4[...], preferred_element_type=jnp.float32)
        h_ref[...] = acc + bm_r[...]
        nb_ref[...] = jnp.dot(edb_r[...], wnb[...],
                              preferred_element_type=jnp.float32) + bnb_r[...]

    g = E_G // BLK
    full = lambda r, c: pl.BlockSpec((r, c), lambda i: (0, 0))
    return pl.pallas_call(
        body,
        grid=(g,),
        in_specs=[
            pl.BlockSpec((BLK, HID), lambda i: (i, 0)),
            pl.BlockSpec((BLK, HID), lambda i: (i + OFF, 0)),
            pl.BlockSpec((BLK, 16), lambda i: (i, 0)),
            pl.BlockSpec((BLK, 4), lambda i: (i, 0)),
            pl.BlockSpec((BLK, 4), lambda i: (i, 0)),
            full(HID, HID), full(HID, HID), full(16, HID), full(4, HID),
            full(1, HID), full(4, HID), full(1, HID),
        ],
        out_specs=[
            pl.BlockSpec((BLK, HID), lambda i: (i, 0)),
            pl.BlockSpec((BLK, HID), lambda i: (i, 0)),
        ],
        out_shape=[
            jax.ShapeDtypeStruct((E_G, HID), jnp.float32),
            jax.ShapeDtypeStruct((E_G, HID), jnp.float32),
        ],
    )(hsd, hsd, ea, xl, edb, W1, W2, W3, W4, bm.reshape(1, HID), Wnb,
      bnb.reshape(1, HID))


def _tc_eb(ea_lg, W, b):
    BLK = 4000

    def body(x_ref, w_ref, b_ref, o_ref):
        o_ref[...] = jnp.dot(x_ref[...], w_ref[...],
                             preferred_element_type=jnp.float32) + b_ref[...]

    return pl.pallas_call(
        body,
        grid=(E_LG // BLK,),
        in_specs=[
            pl.BlockSpec((BLK, 4), lambda i: (i, 0)),
            pl.BlockSpec((4, HID), lambda i: (0, 0)),
            pl.BlockSpec((1, HID), lambda i: (0, 0)),
        ],
        out_specs=pl.BlockSpec((BLK, HID), lambda i: (i, 0)),
        out_shape=jax.ShapeDtypeStruct((E_LG, HID), jnp.float32),
    )(ea_lg, W, b.reshape(1, HID))


def _tc_edgewise(hE, nbE, ebE, tl):
    BLK = 1920
    SB = BLK // 128

    def body(h_r, nb_r, eb_r, t_r, o_ref):
        m = jnp.maximum(h_r[...] * nb_r[...] + eb_r[...], 0.0) + 1e-7
        ex = jnp.exp(jnp.minimum(m * t_r[...], 60.0))
        o_ref[...] = jnp.concatenate(
            [ex.reshape(SB, 128, HID), (m * ex).reshape(SB, 128, HID)],
            axis=1).reshape(2 * BLK, HID)

    return pl.pallas_call(
        body,
        grid=(E_LG // BLK,),
        in_specs=[
            pl.BlockSpec((BLK, HID), lambda i: (i, 0)),
            pl.BlockSpec((BLK, HID), lambda i: (i, 0)),
            pl.BlockSpec((BLK, HID), lambda i: (i, 0)),
            pl.BlockSpec((1, HID), lambda i: (0, 0)),
        ],
        out_specs=pl.BlockSpec((2 * BLK, HID), lambda i: (i, 0)),
        out_shape=jax.ShapeDtypeStruct((2 * E_LG, HID), jnp.float32),
    )(hE, nbE, ebE, tl)


def _tc_layer_matmul(h2, aggr, W, b, hprev):
    BLK = 2000
    residual = hprev is not None

    def body(*refs):
        if residual:
            h2_r, ag_r, w_r, b_r, hp_r, o_ref, st_ref = refs
        else:
            h2_r, ag_r, w_r, b_r, o_ref, st_ref = refs
        acc = jnp.dot(h2_r[...] + ag_r[...], w_r[...],
                      preferred_element_type=jnp.float32) + b_r[...]
        if residual:
            acc = acc + hp_r[...]
        o_ref[...] = acc

        @pl.when(pl.program_id(0) == 0)
        def _():
            st_ref[...] = jnp.zeros_like(st_ref)

        st_ref[...] += jnp.stack([jnp.sum(acc, 0), jnp.sum(acc * acc, 0)])

    ins = [h2, aggr, W, b.reshape(1, HID)] + ([hprev] if residual else [])
    in_specs = [
        pl.BlockSpec((BLK, HID), lambda i: (i, 0)),
        pl.BlockSpec((BLK, HID), lambda i: (i, 0)),
        pl.BlockSpec((HID, HID), lambda i: (0, 0)),
        pl.BlockSpec((1, HID), lambda i: (0, 0)),
    ] + ([pl.BlockSpec((BLK, HID), lambda i: (i, 0))] if residual else [])
    return pl.pallas_call(
        body,
        grid=(N_LG // BLK,),
        in_specs=in_specs,
        out_specs=[
            pl.BlockSpec((BLK, HID), lambda i: (i, 0)),
            pl.BlockSpec((2, HID), lambda i: (0, 0)),
        ],
        out_shape=[
            jax.ShapeDtypeStruct((N_LG, HID), jnp.float32),
            jax.ShapeDtypeStruct((2, HID), jnp.float32),
        ],
    )(*ins)


def _tc_norm_act(h, scale, shift, relu):
    BLK = 2000

    def body(h_r, sc_r, sh_r, o_ref):
        v = h_r[...] * sc_r[...] + sh_r[...]
        o_ref[...] = jnp.maximum(v, 0.0) if relu else v

    return pl.pallas_call(
        body,
        grid=(N_LG // BLK,),
        in_specs=[
            pl.BlockSpec((BLK, HID), lambda i: (i, 0)),
            pl.BlockSpec((1, HID), lambda i: (0, 0)),
            pl.BlockSpec((1, HID), lambda i: (0, 0)),
        ],
        out_specs=pl.BlockSpec((BLK, HID), lambda i: (i, 0)),
        out_shape=jax.ShapeDtypeStruct((N_LG, HID), jnp.float32),
    )(h, scale, shift)


def _tc_pool_pred(parts, batch_r, Wp_pad):
    BLK = 1000
    NGR = 104

    def body(p_r, b_r, wp_r, s_ref, c_ref, o_ref):
        bid = b_r[0, 0, :]
        rows = lax.broadcasted_iota(jnp.int32, (NGR, BLK), 0)
        onehot = (bid[None, :] == rows).astype(jnp.float32)
        emb = p_r[0] + p_r[1]

        @pl.when(pl.program_id(0) == 0)
        def _():
            s_ref[...] = jnp.zeros_like(s_ref)
            c_ref[...] = jnp.zeros_like(c_ref)

        s_ref[...] += jnp.dot(onehot, emb, preferred_element_type=jnp.float32)
        c_ref[...] += jnp.dot(onehot, jnp.ones((BLK, HID), jnp.float32),
                              preferred_element_type=jnp.float32)

        @pl.when(pl.program_id(0) == N_G // BLK - 1)
        def _():
            hg = s_ref[...] / jnp.maximum(c_ref[...], 1.0)
            o_ref[...] = jnp.dot(hg, wp_r[...],
                                 preferred_element_type=jnp.float32)

    return pl.pallas_call(
        body,
        grid=(N_G // BLK,),
        in_specs=[
            pl.BlockSpec((2, BLK, HID), lambda i: (0, i, 0)),
            pl.BlockSpec((1, 1, BLK), lambda i: (i, 0, 0)),
            pl.BlockSpec((HID, 8), lambda i: (0, 0)),
        ],
        out_specs=[
            pl.BlockSpec((NGR, HID), lambda i: (0, 0)),
            pl.BlockSpec((NGR, HID), lambda i: (0, 0)),
            pl.BlockSpec((NGR, 8), lambda i: (0, 0)),
        ],
        out_shape=[
            jax.ShapeDtypeStruct((NGR, HID), jnp.float32),
            jax.ShapeDtypeStruct((NGR, HID), jnp.float32),
            jax.ShapeDtypeStruct((NGR, 8), jnp.float32),
        ],
    )(parts, batch_r, Wp_pad)[2]


def kernel(x_g, edge_index_g, edge_attr_g, x_lg, edge_index_lg, edge_dist_basis,
           edge_attr_lg, batch, W_enc, b_enc, W_msg, b_msg, W_nb, b_nb, W_eb,
           b_eb, W_mlp, b_mlp, gamma, beta, t, W_pred, b_pred):
    src_g, dst_g = edge_index_g[0], edge_index_g[1]
    src, dst = edge_index_lg[0], edge_index_lg[1]

    iota = jnp.arange(E_LG, dtype=jnp.int32)
    dst_s, src_s, perm = lax.sort([dst, src, iota], num_keys=1)
    bounds = jnp.arange(N_CHUNK + 1, dtype=jnp.int32) * D_CHUNK
    ofs = jnp.searchsorted(dst_s, bounds).astype(jnp.int32)
    ofs_pad = jnp.zeros((96,), jnp.int32).at[:N_CHUNK + 1].set(ofs)

    h0 = _tc_enc(x_g, W_enc, b_enc)
    hsd = _sc_gather(h0, jnp.concatenate([src_g, dst_g]))
    W1 = W_msg[:HID]
    W2 = W_msg[HID:2 * HID]
    W3 = W_msg[2 * HID:2 * HID + 16]
    W4 = W_msg[2 * HID + 16:]
    h, nb = _tc_msg(hsd, edge_attr_g, x_lg, edge_dist_basis,
                    W1, W2, W3, W4, b_msg, W_nb, b_nb)
    eb_u = _tc_eb(edge_attr_lg, W_eb, b_eb)

    nbE = _sc_gather(nb, src_s)
    ebE = _sc_gather(eb_u, perm)

    h2 = h
    hprev = None
    hF = None
    for l in range(L):
        hE = _sc_gather(h2, src_s)
        tl = jnp.full((1, HID), t[l], jnp.float32)
        ema = _tc_edgewise(hE, nbE, ebE, tl)
        aggr = _sc_scatter_softmax(ema, dst_s, ofs_pad)
        h_new, st = _tc_layer_matmul(h2, aggr, W_mlp[l], b_mlp[l], hprev)
        mu = st[0] / N_LG
        var = st[1] / N_LG - mu * mu
        sc = gamma[l] / jnp.sqrt(var + 1e-5)
        sh = beta[l] - mu * sc
        if l < L - 1:
            h2 = _tc_norm_act(h_new, sc.reshape(1, HID), sh.reshape(1, HID),
                              relu=True)
        else:
            hF = _tc_norm_act(h_new, sc.reshape(1, HID), sh.reshape(1, HID),
                              relu=False)
        hprev = h_new

    parts = _sc_scatter_final(hF, dst_g)
    batch_r = batch.astype(jnp.int32).reshape(10, 1, 1000)
    Wp_pad = jnp.zeros((HID, 8), jnp.float32).at[:, :1].set(W_pred)
    out = _tc_pool_pred(parts, batch_r, Wp_pad)
    return out[:NUM_GRAPHS, :1] + b_pred[None, :]


# speedup vs baseline: 1.0049x; 1.0049x over previous
import functools

import jax
import jax.numpy as jnp
from jax import lax
from jax.experimental import pallas as pl
from jax.experimental.pallas import tpu as pltpu
from jax.experimental.pallas import tpu_sc as plsc

N_G = 10000
E_G = 160000
E_LG = 480000
N_LG = 160000
HID = 128
L = 7
NUM_GRAPHS = 100

NW = 32
D_CHUNK = 2048
N_CHUNK = 80
ACC_R = D_CHUNK + 128
NGP = 10240

_mesh = plsc.VectorSubcoreMesh(core_axis_name="c", subcore_axis_name="s")


def _sc_gather(table, idx):
    B = idx.shape[0]
    per_w = B // NW
    assert per_w * NW == B and per_w % 8 == 0
    nblk = -(-per_w // 128)
    assert nblk >= 8
    Bp = B + nblk * 128 - per_w
    idx_p = jnp.zeros((Bp,), jnp.int32).at[:B].set(idx)
    NBUF = 6
    ngrp = nblk // NBUF

    @functools.partial(
        pl.kernel, mesh=_mesh,
        out_type=jax.ShapeDtypeStruct((Bp, HID), jnp.float32),
        scratch_types=[
            pltpu.VMEM((nblk * 128,), jnp.int32),
        ] + [pltpu.VMEM((128, HID), jnp.float32) for _ in range(NBUF)]
          + [pltpu.SemaphoreType.DMA for _ in range(2 * NBUF)],
    )
    def k(tab, ix, out, ixall, *bufs_sems):
        rows = bufs_sems[:NBUF]
        gsem = bufs_sems[NBUF:2 * NBUF]
        osem = bufs_sems[2 * NBUF:]
        wid = lax.axis_index("s") * 2 + lax.axis_index("c")
        base = wid * per_w
        pltpu.sync_copy(ix.at[pl.ds(base, nblk * 128)], ixall)

        def g_start(i, b):
            pltpu.async_copy(tab.at[ixall.at[pl.ds(i * 128, 128)]],
                             rows[b], gsem[b])

        def w_start(i, b):
            pltpu.async_copy(rows[b], out.at[pl.ds(base + i * 128, 128)],
                             osem[b])

        g_wait = [
            pltpu.make_async_copy(tab.at[ixall.at[pl.ds(0, 128)]],
                                  rows[b], gsem[b])
            for b in range(NBUF)
        ]
        o_wait = [
            pltpu.make_async_copy(rows[b], out.at[pl.ds(0, 128)], osem[b])
            for b in range(NBUF)
        ]

        for b in range(NBUF - 2):
            g_start(b, b)

        def step(j, b, bprev, traced):
            g_wait[b].wait()
            w_start(j, b)
            k2 = j + NBUF - 2
            if traced:
                @pl.when(k2 < nblk)
                def _():
                    @pl.when(j >= 2)
                    def _():
                        o_wait[bprev].wait()

                    g_start(k2, bprev)
            else:
                if k2 < nblk:
                    if j >= 2:
                        o_wait[bprev].wait()
                    g_start(k2, bprev)

        def grp(j, carry):
            for b in range(NBUF):
                i = j * NBUF + b
                step(i, b, (b - 2) % NBUF, True)
            return carry

        lax.fori_loop(0, ngrp, grp, 0)
        for i in range(ngrp * NBUF, nblk):
            step(i, i % NBUF, (i - 2) % NBUF, False)
        for i in range(nblk - NBUF, nblk):
            o_wait[i % NBUF].wait()

    return k(table, idx_p)


def _sc_scatter_softmax(ema, dst_s, ofs_pad):
    ROWS_T = ACC_R // 16
    FIN_T = D_CHUNK // 16

    @functools.partial(
        pl.kernel, mesh=_mesh,
        out_type=jax.ShapeDtypeStruct((N_LG, HID), jnp.float32),
        scratch_types=[
            pltpu.VMEM((128,), jnp.int32),
            pltpu.VMEM((128,), jnp.int32),
            pltpu.VMEM((128,), jnp.int32),
            pltpu.VMEM((128,), jnp.int32),
            pltpu.VMEM((128, HID), jnp.float32),
            pltpu.VMEM((128, HID), jnp.float32),
            pltpu.VMEM((128, HID), jnp.float32),
            pltpu.VMEM((128, HID), jnp.float32),
            pltpu.VMEM((64, HID), jnp.float32),
            pltpu.VMEM((96,), jnp.int32),
            pltpu.VMEM_SHARED((ACC_R, HID), jnp.float32),
            pltpu.VMEM_SHARED((ACC_R, HID), jnp.float32),
            pltpu.SemaphoreType.DMA,
            pltpu.SemaphoreType.DMA,
        ],
    )
    def k(ema_h, dst_h, ofs_h, out_h,
          draw0, draw1, dloc0, dloc1, pe0, pe1, pme0, pme1, zb, ofsv,
          acc_s, acc_a, sem0, sem1):
        cid = lax.axis_index("c")
        sid = lax.axis_index("s")
        draw = (draw0, draw1)
        dloc = (dloc0, dloc1)
        pe = (pe0, pe1)
        pme = (pme0, pme1)
        psem = (sem0, sem1)
        pltpu.sync_copy(ofs_h, ofsv)

        zero16 = jnp.zeros((16,), jnp.float32)

        def zrow(r, carry):
            for k2 in range(8):
                zb[r, pl.ds(k2 * 16, 16)] = zero16
            return carry

        lax.fori_loop(0, 64, zrow, 0)

        pwait = [
            (pltpu.make_async_copy(
                ema_h.at[pl.ds(0, 128), pl.ds(0, HID)], pe[b], psem[b]),
             pltpu.make_async_copy(
                ema_h.at[pl.ds(0, 128), pl.ds(HID, HID)], pme[b], psem[b]))
            for b in range(2)
        ]

        def chunk_body(c, carry):
            z0 = sid * ROWS_T
            for acc in (acc_s, acc_a):
                pltpu.sync_copy(zb, acc.at[pl.ds(z0, 64)])
                pltpu.sync_copy(zb, acc.at[pl.ds(z0 + 64, 64)])
                pltpu.sync_copy(zb.at[pl.ds(0, ROWS_T - 128)],
                                acc.at[pl.ds(z0 + 128, ROWS_T - 128)])
            plsc.subcore_barrier()

            ov = ofsv[pl.ds(c, 16)]
            start = ov[0]
            end = ov[1]
            start_al = (start // 128) * 128
            nblk = (end - start_al + 127) // 128
            nmine = jnp.maximum((nblk - sid + 15) // 16, 0)
            cbase = c * D_CHUNK

            def pay_start(i, b):
                eofs = start_al + (sid + i * 16) * 128
                pltpu.async_copy(ema_h.at[pl.ds(eofs, 128), pl.ds(0, HID)],
                                 pe[b], psem[b])
                pltpu.async_copy(ema_h.at[pl.ds(eofs, 128), pl.ds(HID, HID)],
                                 pme[b], psem[b])

            for b in range(2):
                @pl.when(b < nmine)
                def _():
                    pay_start(b, b)

            def blk_body(j, carry2):
                for b in range(2):
                    i = j * 2 + b

                    @pl.when(i < nmine)
                    def _():
                        eofs = start_al + (sid + i * 16) * 128
                        pltpu.sync_copy(dst_h.at[pl.ds(eofs, 128)], draw[b])
                        for k2 in range(8):
                            v = draw[b][pl.ds(k2 * 16, 16)]
                            lv = v - cbase
                            ok = (lv >= 0) & (lv < D_CHUNK)
                            dloc[b][pl.ds(k2 * 16, 16)] = jnp.where(
                                ok, lv, D_CHUNK)
                        pwait[b][0].wait()
                        pwait[b][1].wait()
                        pltpu.sync_copy(pe[b], acc_s.at[dloc[b]], add=True)
                        pltpu.sync_copy(pme[b], acc_a.at[dloc[b]], add=True)

                        @pl.when(i + 2 < nmine)
                        def _():
                            pay_start(i + 2, b)
                return carry2

            lax.fori_loop(0, (nmine + 1) // 2, blk_body, 0)
            plsc.subcore_barrier()

            for half in range(1):
                r0 = sid * FIN_T + half * 128

                @pl.when(cbase + r0 < N_LG)
                def _():
                    pltpu.sync_copy(acc_s.at[pl.ds(r0, 128)], pe0)
                    pltpu.sync_copy(acc_a.at[pl.ds(r0, 128)], pme0)

                    def fin(r, carry3):
                        for k2 in range(8):
                            s_ = pe0[r, pl.ds(k2 * 16, 16)]
                            a = pme0[r, pl.ds(k2 * 16, 16)]
                            pe0[r, pl.ds(k2 * 16, 16)] = a / (s_ + 1e-16)
                        return carry3

                    lax.fori_loop(0, 128, fin, 0)
                    pltpu.sync_copy(pe0, out_h.at[pl.ds(cbase + r0, 128)])
            plsc.subcore_barrier()
            return carry

        lax.fori_loop(cid * (N_CHUNK // 2), (cid + 1) * (N_CHUNK // 2),
                      chunk_body, 0)

    return k(ema, dst_s, ofs_pad)


def _sc_scatter_final(h, dst_g):
    per_w = E_G // NW
    nfull = per_w // 128
    tail = per_w - nfull * 128
    ROWS_T = NGP // 16

    @functools.partial(
        pl.kernel, mesh=_mesh,
        out_type=jax.ShapeDtypeStruct((2, NGP, HID), jnp.float32),
        scratch_types=[
            pltpu.VMEM((128,), jnp.int32),
            pltpu.VMEM((128, HID), jnp.float32),
            pltpu.VMEM((128, HID), jnp.float32),
            pltpu.VMEM_SHARED((NGP, HID), jnp.float32),
            pltpu.SemaphoreType.DMA,
        ],
    )
    def k(h_h, dg_h, out_h, draw, pay, zb, acc, sem):
        cid = lax.axis_index("c")
        sid = lax.axis_index("s")
        wid = sid * 2 + cid
        base = wid * per_w

        zero16 = jnp.zeros((16,), jnp.float32)
        trash16 = jnp.full((16,), NGP - 1, jnp.int32)

        def zrow(r, carry):
            for k2 in range(8):
                zb[r, pl.ds(k2 * 16, 16)] = zero16
            return carry

        lax.fori_loop(0, 128, zrow, 0)

        z0 = sid * ROWS_T
        for q in range(5):
            pltpu.sync_copy(zb.at[pl.ds(0, 128)],
                            acc.at[pl.ds(z0 + q * 128, 128)])
        plsc.subcore_barrier()

        def body(i, carry):
            off = base + i * 128
            pltpu.sync_copy(dg_h.at[pl.ds(off, 128)], draw)
            pltpu.sync_copy(h_h.at[pl.ds(off, 128)], pay)
            pltpu.sync_copy(pay, acc.at[draw], add=True)
            return carry

        lax.fori_loop(0, nfull, body, 0)

        for k2 in range(8):
            draw[pl.ds(k2 * 16, 16)] = trash16

        def zpay(r, carry):
            for k2 in range(8):
                pay[r, pl.ds(k2 * 16, 16)] = zero16
            return carry

        lax.fori_loop(0, 128, zpay, 0)
        toff = base + nfull * 128
        pltpu.sync_copy(dg_h.at[pl.ds(toff, tail)], draw.at[pl.ds(0, tail)])
        pltpu.sync_copy(h_h.at[pl.ds(toff, tail)], pay.at[pl.ds(0, tail)])
        pltpu.sync_copy(pay, acc.at[draw], add=True)
        plsc.subcore_barrier()

        for q in range(5):
            pltpu.sync_copy(acc.at[pl.ds(z0 + q * 128, 128)],
                            out_h.at[cid, pl.ds(z0 + q * 128, 128)])

    return k(h, dst_g)


def _tc_enc(x, W, b):
    def body(x_ref, w_ref, b_ref, o_ref):
        o_ref[...] = jnp.dot(x_ref[...], w_ref[...],
                             preferred_element_type=jnp.float32) + b_ref[...]

    return pl.pallas_call(
        body,
        grid=(10,),
        in_specs=[
            pl.BlockSpec((1000, HID), lambda i: (i, 0)),
            pl.BlockSpec((HID, HID), lambda i: (0, 0)),
            pl.BlockSpec((1, HID), lambda i: (0, 0)),
        ],
        out_specs=pl.BlockSpec((1000, HID), lambda i: (i, 0)),
        out_shape=jax.ShapeDtypeStruct((N_G, HID), jnp.float32),
    )(x, W, b.reshape(1, HID))


def _tc_msg(hsd, ea, xl, edb, W1, W2, W3, W4, bm, Wnb, bnb):
    BLK = 2000
    OFF = E_G // BLK

    def body(hs_r, hd_r, ea_r, xl_r, edb_r, w1, w2, w3, w4, bm_r, wnb, bnb_r,
             h_ref, nb_ref):
        acc = jnp.dot(hs_r[...], w1[...], preferred_element_type=jnp.float32)
        acc += jnp.dot(hd_r[...], w2[...], preferred_element_type=jnp.float32)
        acc += jnp.dot(ea_r[...], w3[...], preferred_element_type=jnp.float32)
        acc += jnp.dot(xl_r[...], w4[...], preferred_element_type=jnp.float32)
        h_ref[...] = acc + bm_r[...]
        nb_ref[...] = jnp.dot(edb_r[...], wnb[...],
                              preferred_element_type=jnp.float32) + bnb_r[...]

    g = E_G // BLK
    full = lambda r, c: pl.BlockSpec((r, c), lambda i: (0, 0))
    return pl.pallas_call(
        body,
        grid=(g,),
        in_specs=[
            pl.BlockSpec((BLK, HID), lambda i: (i, 0)),
            pl.BlockSpec((BLK, HID), lambda i: (i + OFF, 0)),
            pl.BlockSpec((BLK, 16), lambda i: (i, 0)),
            pl.BlockSpec((BLK, 4), lambda i: (i, 0)),
            pl.BlockSpec((BLK, 4), lambda i: (i, 0)),
            full(HID, HID), full(HID, HID), full(16, HID), full(4, HID),
            full(1, HID), full(4, HID), full(1, HID),
        ],
        out_specs=[
            pl.BlockSpec((BLK, HID), lambda i: (i, 0)),
            pl.BlockSpec((BLK, HID), lambda i: (i, 0)),
        ],
        out_shape=[
            jax.ShapeDtypeStruct((E_G, HID), jnp.float32),
            jax.ShapeDtypeStruct((E_G, HID), jnp.float32),
        ],
    )(hsd, hsd, ea, xl, edb, W1, W2, W3, W4, bm.reshape(1, HID), Wnb,
      bnb.reshape(1, HID))


def _tc_eb(ea_lg, W, b):
    BLK = 4000

    def body(x_ref, w_ref, b_ref, o_ref):
        o_ref[...] = jnp.dot(x_ref[...], w_ref[...],
                             preferred_element_type=jnp.float32) + b_ref[...]

    return pl.pallas_call(
        body,
        grid=(E_LG // BLK,),
        in_specs=[
            pl.BlockSpec((BLK, 4), lambda i: (i, 0)),
            pl.BlockSpec((4, HID), lambda i: (0, 0)),
            pl.BlockSpec((1, HID), lambda i: (0, 0)),
        ],
        out_specs=pl.BlockSpec((BLK, HID), lambda i: (i, 0)),
        out_shape=jax.ShapeDtypeStruct((E_LG, HID), jnp.float32),
    )(ea_lg, W, b.reshape(1, HID))


def _tc_edgewise(hE, nbE, ebE, tl):
    BLK = 2400

    def body(h_r, nb_r, eb_r, t_r, o_ref):
        m = jnp.maximum(h_r[...] * nb_r[...] + eb_r[...], 0.0) + 1e-7
        ex = jnp.exp(jnp.minimum(m * t_r[...], 60.0))
        o_ref[...] = jnp.concatenate([ex, m * ex], axis=1)

    return pl.pallas_call(
        body,
        grid=(E_LG // BLK,),
        in_specs=[
            pl.BlockSpec((BLK, HID), lambda i: (i, 0)),
            pl.BlockSpec((BLK, HID), lambda i: (i, 0)),
            pl.BlockSpec((BLK, HID), lambda i: (i, 0)),
            pl.BlockSpec((1, HID), lambda i: (0, 0)),
        ],
        out_specs=pl.BlockSpec((BLK, 2 * HID), lambda i: (i, 0)),
        out_shape=jax.ShapeDtypeStruct((E_LG, 2 * HID), jnp.float32),
    )(hE, nbE, ebE, tl)


def _tc_layer_matmul(h2, aggr, W, b, hprev):
    BLK = 2000
    residual = hprev is not None

    def body(*refs):
        if residual:
            h2_r, ag_r, w_r, b_r, hp_r, o_ref, st_ref = refs
        else:
            h2_r, ag_r, w_r, b_r, o_ref, st_ref = refs
        acc = jnp.dot(h2_r[...] + ag_r[...], w_r[...],
                      preferred_element_type=jnp.float32) + b_r[...]
        if residual:
            acc = acc + hp_r[...]
        o_ref[...] = acc

        @pl.when(pl.program_id(0) == 0)
        def _():
            st_ref[...] = jnp.zeros_like(st_ref)

        st_ref[...] += jnp.stack([jnp.sum(acc, 0), jnp.sum(acc * acc, 0)])

    ins = [h2, aggr, W, b.reshape(1, HID)] + ([hprev] if residual else [])
    in_specs = [
        pl.BlockSpec((BLK, HID), lambda i: (i, 0)),
        pl.BlockSpec((BLK, HID), lambda i: (i, 0)),
        pl.BlockSpec((HID, HID), lambda i: (0, 0)),
        pl.BlockSpec((1, HID), lambda i: (0, 0)),
    ] + ([pl.BlockSpec((BLK, HID), lambda i: (i, 0))] if residual else [])
    return pl.pallas_call(
        body,
        grid=(N_LG // BLK,),
        in_specs=in_specs,
        out_specs=[
            pl.BlockSpec((BLK, HID), lambda i: (i, 0)),
            pl.BlockSpec((2, HID), lambda i: (0, 0)),
        ],
        out_shape=[
            jax.ShapeDtypeStruct((N_LG, HID), jnp.float32),
            jax.ShapeDtypeStruct((2, HID), jnp.float32),
        ],
    )(*ins)


def _tc_norm_act(h, scale, shift, relu):
    BLK = 2000

    def body(h_r, sc_r, sh_r, o_ref):
        v = h_r[...] * sc_r[...] + sh_r[...]
        o_ref[...] = jnp.maximum(v, 0.0) if relu else v

    return pl.pallas_call(
        body,
        grid=(N_LG // BLK,),
        in_specs=[
            pl.BlockSpec((BLK, HID), lambda i: (i, 0)),
            pl.BlockSpec((1, HID), lambda i: (0, 0)),
            pl.BlockSpec((1, HID), lambda i: (0, 0)),
        ],
        out_specs=pl.BlockSpec((BLK, HID), lambda i: (i, 0)),
        out_shape=jax.ShapeDtypeStruct((N_LG, HID), jnp.float32),
    )(h, scale, shift)


def _tc_pool_pred(parts, batch_r, Wp_pad):
    BLK = 1000
    NGR = 104

    def body(p_r, b_r, wp_r, s_ref, c_ref, o_ref):
        bid = b_r[0, 0, :]
        rows = lax.broadcasted_iota(jnp.int32, (NGR, BLK), 0)
        onehot = (bid[None, :] == rows).astype(jnp.float32)
        emb = p_r[0] + p_r[1]

        @pl.when(pl.program_id(0) == 0)
        def _():
            s_ref[...] = jnp.zeros_like(s_ref)
            c_ref[...] = jnp.zeros_like(c_ref)

        s_ref[...] += jnp.dot(onehot, emb, preferred_element_type=jnp.float32)
        c_ref[...] += jnp.dot(onehot, jnp.ones((BLK, HID), jnp.float32),
                              preferred_element_type=jnp.float32)

        @pl.when(pl.program_id(0) == N_G // BLK - 1)
        def _():
            hg = s_ref[...] / jnp.maximum(c_ref[...], 1.0)
            o_ref[...] = jnp.dot(hg, wp_r[...],
                                 preferred_element_type=jnp.float32)

    return pl.pallas_call(
        body,
        grid=(N_G // BLK,),
        in_specs=[
            pl.BlockSpec((2, BLK, HID), lambda i: (0, i, 0)),
            pl.BlockSpec((1, 1, BLK), lambda i: (i, 0, 0)),
            pl.BlockSpec((HID, 8), lambda i: (0, 0)),
        ],
        out_specs=[
            pl.BlockSpec((NGR, HID), lambda i: (0, 0)),
            pl.BlockSpec((NGR, HID), lambda i: (0, 0)),
            pl.BlockSpec((NGR, 8), lambda i: (0, 0)),
        ],
        out_shape=[
            jax.ShapeDtypeStruct((NGR, HID), jnp.float32),
            jax.ShapeDtypeStruct((NGR, HID), jnp.float32),
            jax.ShapeDtypeStruct((NGR, 8), jnp.float32),
        ],
    )(parts, batch_r, Wp_pad)[2]


def kernel(x_g, edge_index_g, edge_attr_g, x_lg, edge_index_lg, edge_dist_basis,
           edge_attr_lg, batch, W_enc, b_enc, W_msg, b_msg, W_nb, b_nb, W_eb,
           b_eb, W_mlp, b_mlp, gamma, beta, t, W_pred, b_pred):
    src_g, dst_g = edge_index_g[0], edge_index_g[1]
    src, dst = edge_index_lg[0], edge_index_lg[1]

    iota = jnp.arange(E_LG, dtype=jnp.int32)
    dst_s, src_s, perm = lax.sort([dst, src, iota], num_keys=1)
    bounds = jnp.arange(N_CHUNK + 1, dtype=jnp.int32) * D_CHUNK
    ofs = jnp.searchsorted(dst_s, bounds).astype(jnp.int32)
    ofs_pad = jnp.zeros((96,), jnp.int32).at[:N_CHUNK + 1].set(ofs)

    h0 = _tc_enc(x_g, W_enc, b_enc)
    hsd = _sc_gather(h0, jnp.concatenate([src_g, dst_g]))
    W1 = W_msg[:HID]
    W2 = W_msg[HID:2 * HID]
    W3 = W_msg[2 * HID:2 * HID + 16]
    W4 = W_msg[2 * HID + 16:]
    h, nb = _tc_msg(hsd, edge_attr_g, x_lg, edge_dist_basis,
                    W1, W2, W3, W4, b_msg, W_nb, b_nb)
    eb_u = _tc_eb(edge_attr_lg, W_eb, b_eb)

    nbE = _sc_gather(nb, src_s)
    ebE = _sc_gather(eb_u, perm)

    h2 = h
    hprev = None
    hF = None
    for l in range(L):
        hE = _sc_gather(h2, src_s)
        tl = jnp.full((1, HID), t[l], jnp.float32)
        ema = _tc_edgewise(hE, nbE, ebE, tl)
        aggr = _sc_scatter_softmax(ema, dst_s, ofs_pad)
        h_new, st = _tc_layer_matmul(h2, aggr, W_mlp[l], b_mlp[l], hprev)
        mu = st[0] / N_LG
        var = st[1] / N_LG - mu * mu
        sc = gamma[l] / jnp.sqrt(var + 1e-5)
        sh = beta[l] - mu * sc
        if l < L - 1:
            h2 = _tc_norm_act(h_new, sc.reshape(1, HID), sh.reshape(1, HID),
                              relu=True)
        else:
            hF = _tc_norm_act(h_new, sc.reshape(1, HID), sh.reshape(1, HID),
                              relu=False)
        hprev = h_new

    parts = _sc_scatter_final(hF, dst_g)
    batch_r = batch.astype(jnp.int32).reshape(10, 1, 1000)
    Wp_pad = jnp.zeros((HID, 8), jnp.float32).at[:, :1].set(W_pred)
    out = _tc_pool_pred(parts, batch_r, Wp_pad)
    return out[:NUM_GRAPHS, :1] + b_pred[None, :]


# speedup vs baseline: 1.0118x; 1.0069x over previous
import functools

import jax
import jax.numpy as jnp
from jax import lax
from jax.experimental import pallas as pl
from jax.experimental.pallas import tpu as pltpu
from jax.experimental.pallas import tpu_sc as plsc

N_G = 10000
E_G = 160000
E_LG = 480000
N_LG = 160000
HID = 128
L = 7
NUM_GRAPHS = 100

NW = 32
D_CHUNK = 2048
N_CHUNK = 80
ACC_R = D_CHUNK + 128
NGP = 10240

_mesh = plsc.VectorSubcoreMesh(core_axis_name="c", subcore_axis_name="s")


def _sc_gather(table, idx):
    B = idx.shape[0]
    per_w = B // NW
    assert per_w * NW == B and per_w % 8 == 0
    nblk = -(-per_w // 128)
    assert nblk >= 8
    Bp = B + nblk * 128 - per_w
    idx_p = jnp.zeros((Bp,), jnp.int32).at[:B].set(idx)
    NBUF = 6
    ngrp = nblk // NBUF

    @functools.partial(
        pl.kernel, mesh=_mesh,
        out_type=jax.ShapeDtypeStruct((Bp, HID), jnp.float32),
        scratch_types=[
            pltpu.VMEM((nblk * 128,), jnp.int32),
        ] + [pltpu.VMEM((128, HID), jnp.float32) for _ in range(NBUF)]
          + [pltpu.SemaphoreType.DMA for _ in range(2 * NBUF)],
    )
    def k(tab, ix, out, ixall, *bufs_sems):
        rows = bufs_sems[:NBUF]
        gsem = bufs_sems[NBUF:2 * NBUF]
        osem = bufs_sems[2 * NBUF:]
        wid = lax.axis_index("s") * 2 + lax.axis_index("c")
        base = wid * per_w
        pltpu.sync_copy(ix.at[pl.ds(base, nblk * 128)], ixall)

        def g_start(i, b):
            pltpu.async_copy(tab.at[ixall.at[pl.ds(i * 128, 128)]],
                             rows[b], gsem[b])

        def w_start(i, b):
            pltpu.async_copy(rows[b], out.at[pl.ds(base + i * 128, 128)],
                             osem[b])

        g_wait = [
            pltpu.make_async_copy(tab.at[ixall.at[pl.ds(0, 128)]],
                                  rows[b], gsem[b])
            for b in range(NBUF)
        ]
        o_wait = [
            pltpu.make_async_copy(rows[b], out.at[pl.ds(0, 128)], osem[b])
            for b in range(NBUF)
        ]

        for b in range(NBUF - 2):
            g_start(b, b)

        def step(j, b, bprev, traced):
            g_wait[b].wait()
            w_start(j, b)
            k2 = j + NBUF - 2
            if traced:
                @pl.when(k2 < nblk)
                def _():
                    @pl.when(j >= 2)
                    def _():
                        o_wait[bprev].wait()

                    g_start(k2, bprev)
            else:
                if k2 < nblk:
                    if j >= 2:
                        o_wait[bprev].wait()
                    g_start(k2, bprev)

        def grp(j, carry):
            for b in range(NBUF):
                i = j * NBUF + b
                step(i, b, (b - 2) % NBUF, True)
            return carry

        lax.fori_loop(0, ngrp, grp, 0)
        for i in range(ngrp * NBUF, nblk):
            step(i, i % NBUF, (i - 2) % NBUF, False)
        for i in range(nblk - NBUF, nblk):
            o_wait[i % NBUF].wait()

    return k(table, idx_p)


def _sc_scatter_softmax(ema, dst_s, ofs_pad):
    ROWS_T = ACC_R // 16
    FIN_T = D_CHUNK // 16

    @functools.partial(
        pl.kernel, mesh=_mesh,
        out_type=jax.ShapeDtypeStruct((N_LG, HID), jnp.float32),
        scratch_types=[
            pltpu.VMEM((128,), jnp.int32),
            pltpu.VMEM((128,), jnp.int32),
            pltpu.VMEM((128,), jnp.int32),
            pltpu.VMEM((128,), jnp.int32),
            pltpu.VMEM((128, HID), jnp.float32),
            pltpu.VMEM((128, HID), jnp.float32),
            pltpu.VMEM((128, HID), jnp.float32),
            pltpu.VMEM((128, HID), jnp.float32),
            pltpu.VMEM((64, HID), jnp.float32),
            pltpu.VMEM((96,), jnp.int32),
            pltpu.VMEM_SHARED((ACC_R, HID), jnp.float32),
            pltpu.VMEM_SHARED((ACC_R, HID), jnp.float32),
            pltpu.SemaphoreType.DMA,
            pltpu.SemaphoreType.DMA,
        ],
    )
    def k(ema_h, dst_h, ofs_h, out_h,
          draw0, draw1, dloc0, dloc1, pe0, pe1, pme0, pme1, zb, ofsv,
          acc_s, acc_a, sem0, sem1):
        cid = lax.axis_index("c")
        sid = lax.axis_index("s")
        draw = (draw0, draw1)
        dloc = (dloc0, dloc1)
        pe = (pe0, pe1)
        pme = (pme0, pme1)
        psem = (sem0, sem1)
        pltpu.sync_copy(ofs_h, ofsv)

        zero16 = jnp.zeros((16,), jnp.float32)

        def zrow(r, carry):
            for k2 in range(8):
                zb[r, pl.ds(k2 * 16, 16)] = zero16
            return carry

        lax.fori_loop(0, 64, zrow, 0)

        pwait = [
            (pltpu.make_async_copy(
                ema_h.at[pl.ds(0, 128), pl.ds(0, HID)], pe[b], psem[b]),
             pltpu.make_async_copy(
                ema_h.at[pl.ds(0, 128), pl.ds(HID, HID)], pme[b], psem[b]))
            for b in range(2)
        ]

        def chunk_body(c, carry):
            z0 = sid * ROWS_T
            for acc in (acc_s, acc_a):
                pltpu.sync_copy(zb, acc.at[pl.ds(z0, 64)])
                pltpu.sync_copy(zb, acc.at[pl.ds(z0 + 64, 64)])
                pltpu.sync_copy(zb.at[pl.ds(0, ROWS_T - 128)],
                                acc.at[pl.ds(z0 + 128, ROWS_T - 128)])
            plsc.subcore_barrier()

            ov = ofsv[pl.ds(c, 16)]
            start = ov[0]
            end = ov[1]
            start_al = (start // 128) * 128
            nblk = (end - start_al + 127) // 128
            nmine = jnp.maximum((nblk - sid + 15) // 16, 0)
            cbase = c * D_CHUNK

            def pay_start(i, b):
                eofs = start_al + (sid + i * 16) * 128
                pltpu.async_copy(ema_h.at[pl.ds(eofs, 128), pl.ds(0, HID)],
                                 pe[b], psem[b])
                pltpu.async_copy(ema_h.at[pl.ds(eofs, 128), pl.ds(HID, HID)],
                                 pme[b], psem[b])

            for b in range(2):
                @pl.when(b < nmine)
                def _():
                    pay_start(b, b)

            def blk_body(j, carry2):
                for b in range(2):
                    i = j * 2 + b

                    @pl.when(i < nmine)
                    def _():
                        eofs = start_al + (sid + i * 16) * 128
                        pltpu.sync_copy(dst_h.at[pl.ds(eofs, 128)], draw[b])
                        for k2 in range(8):
                            v = draw[b][pl.ds(k2 * 16, 16)]
                            lv = v - cbase
                            ok = (lv >= 0) & (lv < D_CHUNK)
                            dloc[b][pl.ds(k2 * 16, 16)] = jnp.where(
                                ok, lv, D_CHUNK)
                        pwait[b][0].wait()
                        pwait[b][1].wait()
                        pltpu.sync_copy(pe[b], acc_s.at[dloc[b]], add=True)
                        pltpu.sync_copy(pme[b], acc_a.at[dloc[b]], add=True)

                        @pl.when(i + 2 < nmine)
                        def _():
                            pay_start(i + 2, b)
                return carry2

            lax.fori_loop(0, (nmine + 1) // 2, blk_body, 0)
            plsc.subcore_barrier()

            for half in range(1):
                r0 = sid * FIN_T + half * 128

                @pl.when(cbase + r0 < N_LG)
                def _():
                    pltpu.sync_copy(acc_s.at[pl.ds(r0, 128)], pe0)
                    pltpu.sync_copy(acc_a.at[pl.ds(r0, 128)], pme0)

                    def fin(r4, carry3):
                        for u in range(4):
                            r = r4 * 4 + u
                            for k2 in range(8):
                                s_ = pe0[r, pl.ds(k2 * 16, 16)]
                                a = pme0[r, pl.ds(k2 * 16, 16)]
                                pe0[r, pl.ds(k2 * 16, 16)] = a / (s_ + 1e-16)
                        return carry3

                    lax.fori_loop(0, 32, fin, 0)
                    pltpu.sync_copy(pe0, out_h.at[pl.ds(cbase + r0, 128)])
            plsc.subcore_barrier()
            return carry

        lax.fori_loop(cid * (N_CHUNK // 2), (cid + 1) * (N_CHUNK // 2),
                      chunk_body, 0)

    return k(ema, dst_s, ofs_pad)


def _sc_scatter_final(h, dst_g):
    per_w = E_G // NW
    nfull = per_w // 128
    tail = per_w - nfull * 128
    ROWS_T = NGP // 16

    @functools.partial(
        pl.kernel, mesh=_mesh,
        out_type=jax.ShapeDtypeStruct((2, NGP, HID), jnp.float32),
        scratch_types=[
            pltpu.VMEM((128,), jnp.int32),
            pltpu.VMEM((128, HID), jnp.float32),
            pltpu.VMEM((128, HID), jnp.float32),
            pltpu.VMEM_SHARED((NGP, HID), jnp.float32),
            pltpu.SemaphoreType.DMA,
        ],
    )
    def k(h_h, dg_h, out_h, draw, pay, zb, acc, sem):
        cid = lax.axis_index("c")
        sid = lax.axis_index("s")
        wid = sid * 2 + cid
        base = wid * per_w

        zero16 = jnp.zeros((16,), jnp.float32)
        trash16 = jnp.full((16,), NGP - 1, jnp.int32)

        def zrow(r, carry):
            for k2 in range(8):
                zb[r, pl.ds(k2 * 16, 16)] = zero16
            return carry

        lax.fori_loop(0, 128, zrow, 0)

        z0 = sid * ROWS_T
        for q in range(5):
            pltpu.sync_copy(zb.at[pl.ds(0, 128)],
                            acc.at[pl.ds(z0 + q * 128, 128)])
        plsc.subcore_barrier()

        def body(i, carry):
            off = base + i * 128
            pltpu.sync_copy(dg_h.at[pl.ds(off, 128)], draw)
            pltpu.sync_copy(h_h.at[pl.ds(off, 128)], pay)
            pltpu.sync_copy(pay, acc.at[draw], add=True)
            return carry

        lax.fori_loop(0, nfull, body, 0)

        for k2 in range(8):
            draw[pl.ds(k2 * 16, 16)] = trash16

        def zpay(r, carry):
            for k2 in range(8):
                pay[r, pl.ds(k2 * 16, 16)] = zero16
            return carry

        lax.fori_loop(0, 128, zpay, 0)
        toff = base + nfull * 128
        pltpu.sync_copy(dg_h.at[pl.ds(toff, tail)], draw.at[pl.ds(0, tail)])
        pltpu.sync_copy(h_h.at[pl.ds(toff, tail)], pay.at[pl.ds(0, tail)])
        pltpu.sync_copy(pay, acc.at[draw], add=True)
        plsc.subcore_barrier()

        for q in range(5):
            pltpu.sync_copy(acc.at[pl.ds(z0 + q * 128, 128)],
                            out_h.at[cid, pl.ds(z0 + q * 128, 128)])

    return k(h, dst_g)


def _tc_enc(x, W, b):
    def body(x_ref, w_ref, b_ref, o_ref):
        o_ref[...] = jnp.dot(x_ref[...], w_ref[...],
                             preferred_element_type=jnp.float32) + b_ref[...]

    return pl.pallas_call(
        body,
        grid=(10,),
        in_specs=[
            pl.BlockSpec((1000, HID), lambda i: (i, 0)),
            pl.BlockSpec((HID, HID), lambda i: (0, 0)),
            pl.BlockSpec((1, HID), lambda i: (0, 0)),
        ],
        out_specs=pl.BlockSpec((1000, HID), lambda i: (i, 0)),
        out_shape=jax.ShapeDtypeStruct((N_G, HID), jnp.float32),
    )(x, W, b.reshape(1, HID))


def _tc_msg(hsd, ea, xl, edb, W1, W2, W3, W4, bm, Wnb, bnb):
    BLK = 2000
    OFF = E_G // BLK

    def body(hs_r, hd_r, ea_r, xl_r, edb_r, w1, w2, w3, w4, bm_r, wnb, bnb_r,
             h_ref, nb_ref):
        acc = jnp.dot(hs_r[...], w1[...], preferred_element_type=jnp.float32)
        acc += jnp.dot(hd_r[...], w2[...], preferred_element_type=jnp.float32)
        acc += jnp.dot(ea_r[...], w3[...], preferred_element_type=jnp.float32)
        acc += jnp.dot(xl_r[...], w4[...], preferred_element_type=jnp.float32)
        h_ref[...] = acc + bm_r[...]
        nb_ref[...] = jnp.dot(edb_r[...], wnb[...],
                              preferred_element_type=jnp.float32) + bnb_r[...]

    g = E_G // BLK
    full = lambda r, c: pl.BlockSpec((r, c), lambda i: (0, 0))
    return pl.pallas_call(
        body,
        grid=(g,),
        in_specs=[
            pl.BlockSpec((BLK, HID), lambda i: (i, 0)),
            pl.BlockSpec((BLK, HID), lambda i: (i + OFF, 0)),
            pl.BlockSpec((BLK, 16), lambda i: (i, 0)),
            pl.BlockSpec((BLK, 4), lambda i: (i, 0)),
            pl.BlockSpec((BLK, 4), lambda i: (i, 0)),
            full(HID, HID), full(HID, HID), full(16, HID), full(4, HID),
            full(1, HID), full(4, HID), full(1, HID),
        ],
        out_specs=[
            pl.BlockSpec((BLK, HID), lambda i: (i, 0)),
            pl.BlockSpec((BLK, HID), lambda i: (i, 0)),
        ],
        out_shape=[
            jax.ShapeDtypeStruct((E_G, HID), jnp.float32),
            jax.ShapeDtypeStruct((E_G, HID), jnp.float32),
        ],
    )(hsd, hsd, ea, xl, edb, W1, W2, W3, W4, bm.reshape(1, HID), Wnb,
      bnb.reshape(1, HID))


def _tc_eb(ea_lg, W, b):
    BLK = 4000

    def body(x_ref, w_ref, b_ref, o_ref):
        o_ref[...] = jnp.dot(x_ref[...], w_ref[...],
                             preferred_element_type=jnp.float32) + b_ref[...]

    return pl.pallas_call(
        body,
        grid=(E_LG // BLK,),
        in_specs=[
            pl.BlockSpec((BLK, 4), lambda i: (i, 0)),
            pl.BlockSpec((4, HID), lambda i: (0, 0)),
            pl.BlockSpec((1, HID), lambda i: (0, 0)),
        ],
        out_specs=pl.BlockSpec((BLK, HID), lambda i: (i, 0)),
        out_shape=jax.ShapeDtypeStruct((E_LG, HID), jnp.float32),
    )(ea_lg, W, b.reshape(1, HID))


def _tc_edgewise(hE, nbE, ebE, tl):
    BLK = 2400

    def body(h_r, nb_r, eb_r, t_r, o_ref):
        m = jnp.maximum(h_r[...] * nb_r[...] + eb_r[...], 0.0) + 1e-7
        ex = jnp.exp(jnp.minimum(m * t_r[...], 60.0))
        o_ref[...] = jnp.concatenate([ex, m * ex], axis=1)

    return pl.pallas_call(
        body,
        grid=(E_LG // BLK,),
        in_specs=[
            pl.BlockSpec((BLK, HID), lambda i: (i, 0)),
            pl.BlockSpec((BLK, HID), lambda i: (i, 0)),
            pl.BlockSpec((BLK, HID), lambda i: (i, 0)),
            pl.BlockSpec((1, HID), lambda i: (0, 0)),
        ],
        out_specs=pl.BlockSpec((BLK, 2 * HID), lambda i: (i, 0)),
        out_shape=jax.ShapeDtypeStruct((E_LG, 2 * HID), jnp.float32),
    )(hE, nbE, ebE, tl)


def _tc_layer_matmul(h2, aggr, W, b, hprev):
    BLK = 2000
    residual = hprev is not None

    def body(*refs):
        if residual:
            h2_r, ag_r, w_r, b_r, hp_r, o_ref, st_ref = refs
        else:
            h2_r, ag_r, w_r, b_r, o_ref, st_ref = refs
        acc = jnp.dot(h2_r[...] + ag_r[...], w_r[...],
                      preferred_element_type=jnp.float32) + b_r[...]
        if residual:
            acc = acc + hp_r[...]
        o_ref[...] = acc

        @pl.when(pl.program_id(0) == 0)
        def _():
            st_ref[...] = jnp.zeros_like(st_ref)

        st_ref[...] += jnp.stack([jnp.sum(acc, 0), jnp.sum(acc * acc, 0)])

    ins = [h2, aggr, W, b.reshape(1, HID)] + ([hprev] if residual else [])
    in_specs = [
        pl.BlockSpec((BLK, HID), lambda i: (i, 0)),
        pl.BlockSpec((BLK, HID), lambda i: (i, 0)),
        pl.BlockSpec((HID, HID), lambda i: (0, 0)),
        pl.BlockSpec((1, HID), lambda i: (0, 0)),
    ] + ([pl.BlockSpec((BLK, HID), lambda i: (i, 0))] if residual else [])
    return pl.pallas_call(
        body,
        grid=(N_LG // BLK,),
        in_specs=in_specs,
        out_specs=[
            pl.BlockSpec((BLK, HID), lambda i: (i, 0)),
            pl.BlockSpec((2, HID), lambda i: (0, 0)),
        ],
        out_shape=[
            jax.ShapeDtypeStruct((N_LG, HID), jnp.float32),
            jax.ShapeDtypeStruct((2, HID), jnp.float32),
        ],
    )(*ins)


def _tc_norm_act(h, scale, shift, relu):
    BLK = 2000

    def body(h_r, sc_r, sh_r, o_ref):
        v = h_r[...] * sc_r[...] + sh_r[...]
        o_ref[...] = jnp.maximum(v, 0.0) if relu else v

    return pl.pallas_call(
        body,
        grid=(N_LG // BLK,),
        in_specs=[
            pl.BlockSpec((BLK, HID), lambda i: (i, 0)),
            pl.BlockSpec((1, HID), lambda i: (0, 0)),
            pl.BlockSpec((1, HID), lambda i: (0, 0)),
        ],
        out_specs=pl.BlockSpec((BLK, HID), lambda i: (i, 0)),
        out_shape=jax.ShapeDtypeStruct((N_LG, HID), jnp.float32),
    )(h, scale, shift)


def _tc_pool_pred(parts, batch_r, Wp_pad):
    BLK = 1000
    NGR = 104

    def body(p_r, b_r, wp_r, s_ref, c_ref, o_ref):
        bid = b_r[0, 0, :]
        rows = lax.broadcasted_iota(jnp.int32, (NGR, BLK), 0)
        onehot = (bid[None, :] == rows).astype(jnp.float32)
        emb = p_r[0] + p_r[1]

        @pl.when(pl.program_id(0) == 0)
        def _():
            s_ref[...] = jnp.zeros_like(s_ref)
            c_ref[...] = jnp.zeros_like(c_ref)

        s_ref[...] += jnp.dot(onehot, emb, preferred_element_type=jnp.float32)
        c_ref[...] += jnp.dot(onehot, jnp.ones((BLK, HID), jnp.float32),
                              preferred_element_type=jnp.float32)

        @pl.when(pl.program_id(0) == N_G // BLK - 1)
        def _():
            hg = s_ref[...] / jnp.maximum(c_ref[...], 1.0)
            o_ref[...] = jnp.dot(hg, wp_r[...],
                                 preferred_element_type=jnp.float32)

    return pl.pallas_call(
        body,
        grid=(N_G // BLK,),
        in_specs=[
            pl.BlockSpec((2, BLK, HID), lambda i: (0, i, 0)),
            pl.BlockSpec((1, 1, BLK), lambda i: (i, 0, 0)),
            pl.BlockSpec((HID, 8), lambda i: (0, 0)),
        ],
        out_specs=[
            pl.BlockSpec((NGR, HID), lambda i: (0, 0)),
            pl.BlockSpec((NGR, HID), lambda i: (0, 0)),
            pl.BlockSpec((NGR, 8), lambda i: (0, 0)),
        ],
        out_shape=[
            jax.ShapeDtypeStruct((NGR, HID), jnp.float32),
            jax.ShapeDtypeStruct((NGR, HID), jnp.float32),
            jax.ShapeDtypeStruct((NGR, 8), jnp.float32),
        ],
    )(parts, batch_r, Wp_pad)[2]


def kernel(x_g, edge_index_g, edge_attr_g, x_lg, edge_index_lg, edge_dist_basis,
           edge_attr_lg, batch, W_enc, b_enc, W_msg, b_msg, W_nb, b_nb, W_eb,
           b_eb, W_mlp, b_mlp, gamma, beta, t, W_pred, b_pred):
    src_g, dst_g = edge_index_g[0], edge_index_g[1]
    src, dst = edge_index_lg[0], edge_index_lg[1]

    iota = jnp.arange(E_LG, dtype=jnp.int32)
    dst_s, src_s, perm = lax.sort([dst, src, iota], num_keys=1)
    bounds = jnp.arange(N_CHUNK + 1, dtype=jnp.int32) * D_CHUNK
    ofs = jnp.searchsorted(dst_s, bounds).astype(jnp.int32)
    ofs_pad = jnp.zeros((96,), jnp.int32).at[:N_CHUNK + 1].set(ofs)

    h0 = _tc_enc(x_g, W_enc, b_enc)
    hsd = _sc_gather(h0, jnp.concatenate([src_g, dst_g]))
    W1 = W_msg[:HID]
    W2 = W_msg[HID:2 * HID]
    W3 = W_msg[2 * HID:2 * HID + 16]
    W4 = W_msg[2 * HID + 16:]
    h, nb = _tc_msg(hsd, edge_attr_g, x_lg, edge_dist_basis,
                    W1, W2, W3, W4, b_msg, W_nb, b_nb)
    eb_u = _tc_eb(edge_attr_lg, W_eb, b_eb)

    nbE = _sc_gather(nb, src_s)
    ebE = _sc_gather(eb_u, perm)

    h2 = h
    hprev = None
    hF = None
    for l in range(L):
        hE = _sc_gather(h2, src_s)
        tl = jnp.full((1, HID), t[l], jnp.float32)
        ema = _tc_edgewise(hE, nbE, ebE, tl)
        aggr = _sc_scatter_softmax(ema, dst_s, ofs_pad)
        h_new, st = _tc_layer_matmul(h2, aggr, W_mlp[l], b_mlp[l], hprev)
        mu = st[0] / N_LG
        var = st[1] / N_LG - mu * mu
        sc = gamma[l] / jnp.sqrt(var + 1e-5)
        sh = beta[l] - mu * sc
        if l < L - 1:
            h2 = _tc_norm_act(h_new, sc.reshape(1, HID), sh.reshape(1, HID),
                              relu=True)
        else:
            hF = _tc_norm_act(h_new, sc.reshape(1, HID), sh.reshape(1, HID),
                              relu=False)
        hprev = h_new

    parts = _sc_scatter_final(hF, dst_g)
    batch_r = batch.astype(jnp.int32).reshape(10, 1, 1000)
    Wp_pad = jnp.zeros((HID, 8), jnp.float32).at[:, :1].set(W_pred)
    out = _tc_pool_pred(parts, batch_r, Wp_pad)
    return out[:NUM_GRAPHS, :1] + b_pred[None, :]
